# baseline (device time: 694794 ns/iter reference)
import jax
import jax.numpy as jnp
from jax import lax
from jax.experimental import pallas as pl
from jax.experimental.pallas import tpu as pltpu


M = 32768
N = 1024
Q = M // 4

ACH = (256, 256, 512, 1024, 2048, 2048, 2048)
AOFF = tuple(sum(ACH[:i]) for i in range(len(ACH)))

DX = ((0, 1536), (1536, 1536))
CY = ((3072, 1024), (4096, 1536))
CZ = ((5632, 512), (6144, 2048))
CY_AFTER = {4: CY[0], 5: CY[1]}
CZ_AFTER = {5: CZ[0], 6: CZ[1]}

N_SEMS = 2 * len(ACH) + len(ACH) + len(DX) + len(CY) + len(CZ)
TILE = 2048


def _body(x_ref, out_ref, rem_ref, recv_sems, send_sems, lsems,
          vx0, vx1, vr0, vr1, vo0, vo1):
    my_x = lax.axis_index("x")
    my_y = lax.axis_index("y")
    my_z = lax.axis_index("z")
    xp = (1 - my_x, my_y, my_z)
    yp = (my_x, 1 - my_y, my_z)
    zp = (my_x, my_y, 1 - my_z)

    q = 2 * my_y + my_z
    qy = 2 * (1 - my_y) + my_z
    qz = 2 * my_y + (1 - my_z)
    d = 2 * (1 - my_y) + (1 - my_z)

    vx = (vx0, vx1)
    vr = (vr0, vr1)
    vo = (vo0, vo1)

    sem_counter = [0]

    def mk(src_base, row, rows, dev):
        i = sem_counter[0]
        sem_counter[0] += 1
        return pltpu.make_async_remote_copy(
            src_ref=src_base.at[pl.ds(row, rows), :],
            dst_ref=rem_ref.at[pl.ds(row, rows), :],
            send_sem=send_sems.at[i],
            recv_sem=recv_sems.at[i],
            device_id=dev,
            device_id_type=pl.DeviceIdType.MESH,
        )

    def add_sweep(tiles, pending_out):
        copies = [None, None]

        def start_in(k):
            row, rows = tiles[k]
            b = k % 2
            cx = pltpu.make_async_copy(
                x_ref.at[pl.ds(row, rows), :],
                vx[b].at[pl.ds(0, rows), :], lsems.at[2 * b])
            cr = pltpu.make_async_copy(
                rem_ref.at[pl.ds(row, rows), :],
                vr[b].at[pl.ds(0, rows), :], lsems.at[2 * b + 1])
            cx.start()
            cr.start()
            copies[b] = (cx, cr)

        start_in(0)
        for k in range(len(tiles)):
            b = k % 2
            if k + 1 < len(tiles):
                start_in(k + 1)
            cx, cr = copies[b]
            cx.wait()
            cr.wait()
            if pending_out[b] is not None:
                pending_out[b].wait()
            row, rows = tiles[k]
            vo[b][:rows, :] = vx[b][:rows, :] + vr[b][:rows, :]
            co = pltpu.make_async_copy(
                vo[b].at[pl.ds(0, rows), :],
                out_ref.at[pl.ds(row, rows), :], lsems.at[4 + b])
            co.start()
            pending_out[b] = co

    A = []
    for c in range(len(ACH)):
        r = mk(x_ref, q * Q + AOFF[c], ACH[c], xp)
        r.start()
        A.append(r)
    Adx = []
    for row, rows in DX:
        r = mk(x_ref, d * Q + row, rows, xp)
        r.start()
        Adx.append(r)

    By, Bz = [], []
    for c in range(len(ACH)):
        A[c].wait_recv()
        ry = mk(rem_ref, q * Q + AOFF[c], ACH[c], yp)
        ry.start()
        By.append(ry)
        rz = mk(rem_ref, q * Q + AOFF[c], ACH[c], zp)
        rz.start()
        Bz.append(rz)

    Cy, Cz = [], []
    for c in range(len(ACH)):
        Bz[c].wait_recv()
        if c in CY_AFTER:
            row, rows = CY_AFTER[c]
            r = mk(rem_ref, qz * Q + row, rows, yp)
            r.start()
            Cy.append(r)
        By[c].wait_recv()
        if c in CZ_AFTER:
            row, rows = CZ_AFTER[c]
            r = mk(rem_ref, qy * Q + row, rows, zp)
            r.start()
            Cz.append(r)

    pending_out = [None, None]
    tiles = []
    for region in (q, qz, qy):
        for t in range(Q // TILE):
            tiles.append((region * Q + t * TILE, TILE))
    add_sweep(tiles, pending_out)

    for r in Adx + Cy + Cz:
        r.wait_recv()
    add_sweep([(d * Q + t * TILE, TILE) for t in range(Q // TILE)],
              pending_out)

    for co in pending_out:
        if co is not None:
            co.wait()
    for r in A + Adx + By + Bz + Cy + Cz:
        r.wait_send()


def kernel(x):
    m, n = x.shape
    assert (m, n) == (M, N)

    out, _rem = pl.pallas_call(
        _body,
        out_shape=[
            jax.ShapeDtypeStruct((m, n), x.dtype),
            jax.ShapeDtypeStruct((m, n), x.dtype),
        ],
        in_specs=[pl.BlockSpec(memory_space=pltpu.MemorySpace.HBM)],
        out_specs=[
            pl.BlockSpec(memory_space=pltpu.MemorySpace.HBM),
            pl.BlockSpec(memory_space=pltpu.MemorySpace.HBM),
        ],
        scratch_shapes=[
            pltpu.SemaphoreType.DMA((N_SEMS,)),
            pltpu.SemaphoreType.DMA((N_SEMS,)),
            pltpu.SemaphoreType.DMA((6,)),
            pltpu.VMEM((TILE, N), x.dtype),
            pltpu.VMEM((TILE, N), x.dtype),
            pltpu.VMEM((TILE, N), x.dtype),
            pltpu.VMEM((TILE, N), x.dtype),
            pltpu.VMEM((TILE, N), x.dtype),
            pltpu.VMEM((TILE, N), x.dtype),
        ],
        compiler_params=pltpu.CompilerParams(
            vmem_limit_bytes=100 * 1024 * 1024,
        ),
    )(x)
    return out


# device time: 692249 ns/iter; 1.0037x vs baseline; 1.0037x over previous
import jax
import jax.numpy as jnp
from jax import lax
from jax.experimental import pallas as pl
from jax.experimental.pallas import tpu as pltpu


M = 32768
N = 1024
Q = M // 4

ACH = (256, 256, 512, 1024, 2048, 2048, 2048)
AOFF = tuple(sum(ACH[:i]) for i in range(len(ACH)))

DX = ((0, 1536), (1536, 1536))
CY = ((3072, 1024), (4096, 1536))
CZ = ((5632, 512), (6144, 2048))
CY_AFTER = {4: CY[0], 5: CY[1]}
CZ_AFTER = {5: CZ[0], 6: CZ[1]}

N_SEMS = 2 * len(ACH) + len(ACH) + len(DX) + len(CY) + len(CZ)
MAX_ADD_ROWS = 3072


def _body(x_ref, out_ref, rem_ref, recv_sems, send_sems, lsems,
          vx_ref, vr_ref, vo_ref):
    my_x = lax.axis_index("x")
    my_y = lax.axis_index("y")
    my_z = lax.axis_index("z")
    xp = (1 - my_x, my_y, my_z)
    yp = (my_x, 1 - my_y, my_z)
    zp = (my_x, my_y, 1 - my_z)

    q = 2 * my_y + my_z
    qy = 2 * (1 - my_y) + my_z
    qz = 2 * my_y + (1 - my_z)
    d = 2 * (1 - my_y) + (1 - my_z)

    sem_counter = [0]

    def mk(src_base, row, rows, dev):
        i = sem_counter[0]
        sem_counter[0] += 1
        return pltpu.make_async_remote_copy(
            src_ref=src_base.at[pl.ds(row, rows), :],
            dst_ref=rem_ref.at[pl.ds(row, rows), :],
            send_sem=send_sems.at[i],
            recv_sem=recv_sems.at[i],
            device_id=dev,
            device_id_type=pl.DeviceIdType.MESH,
        )

    def add_chunk(row, rows):
        cx = pltpu.make_async_copy(
            x_ref.at[pl.ds(row, rows), :], vx_ref.at[pl.ds(0, rows), :],
            lsems.at[0])
        cr = pltpu.make_async_copy(
            rem_ref.at[pl.ds(row, rows), :], vr_ref.at[pl.ds(0, rows), :],
            lsems.at[1])
        cx.start()
        cr.start()
        cx.wait()
        cr.wait()
        vo_ref[:rows, :] = vx_ref[:rows, :] + vr_ref[:rows, :]
        co = pltpu.make_async_copy(
            vo_ref.at[pl.ds(0, rows), :], out_ref.at[pl.ds(row, rows), :],
            lsems.at[2])
        co.start()
        co.wait()

    barrier_sem = pltpu.get_barrier_semaphore()
    for nbr in (xp, yp, zp):
        pl.semaphore_signal(
            barrier_sem, inc=1, device_id=nbr,
            device_id_type=pl.DeviceIdType.MESH)
    pl.semaphore_wait(barrier_sem, 3)

    A = []
    for c in range(len(ACH)):
        r = mk(x_ref, q * Q + AOFF[c], ACH[c], xp)
        r.start()
        A.append(r)
    Adx = []
    for row, rows in DX:
        r = mk(x_ref, d * Q + row, rows, xp)
        r.start()
        Adx.append(r)

    By, Bz = [], []
    for c in range(len(ACH)):
        A[c].wait_recv()
        ry = mk(rem_ref, q * Q + AOFF[c], ACH[c], yp)
        ry.start()
        By.append(ry)
        rz = mk(rem_ref, q * Q + AOFF[c], ACH[c], zp)
        rz.start()
        Bz.append(rz)
        add_chunk(q * Q + AOFF[c], ACH[c])

    Cy, Cz = [], []
    for c in range(len(ACH)):
        Bz[c].wait_recv()
        if c in CY_AFTER:
            row, rows = CY_AFTER[c]
            r = mk(rem_ref, qz * Q + row, rows, yp)
            r.start()
            Cy.append(r)
        By[c].wait_recv()
        if c in CZ_AFTER:
            row, rows = CZ_AFTER[c]
            r = mk(rem_ref, qy * Q + row, rows, zp)
            r.start()
            Cz.append(r)
        add_chunk(qz * Q + AOFF[c], ACH[c])
        add_chunk(qy * Q + AOFF[c], ACH[c])

    for i in range(2):
        Cy[i].wait_recv()
        add_chunk(d * Q + CY[i][0], CY[i][1])
        Cz[i].wait_recv()
        add_chunk(d * Q + CZ[i][0], CZ[i][1])
    for i, (row, rows) in enumerate(DX):
        Adx[i].wait_recv()
        add_chunk(d * Q + row, rows)

    for r in A + Adx + By + Bz + Cy + Cz:
        r.wait_send()


def kernel(x):
    m, n = x.shape
    assert (m, n) == (M, N)

    out, _rem = pl.pallas_call(
        _body,
        out_shape=[
            jax.ShapeDtypeStruct((m, n), x.dtype),
            jax.ShapeDtypeStruct((m, n), x.dtype),
        ],
        in_specs=[pl.BlockSpec(memory_space=pltpu.MemorySpace.HBM)],
        out_specs=[
            pl.BlockSpec(memory_space=pltpu.MemorySpace.HBM),
            pl.BlockSpec(memory_space=pltpu.MemorySpace.HBM),
        ],
        scratch_shapes=[
            pltpu.SemaphoreType.DMA((N_SEMS,)),
            pltpu.SemaphoreType.DMA((N_SEMS,)),
            pltpu.SemaphoreType.DMA((3,)),
            pltpu.VMEM((MAX_ADD_ROWS, N), x.dtype),
            pltpu.VMEM((MAX_ADD_ROWS, N), x.dtype),
            pltpu.VMEM((MAX_ADD_ROWS, N), x.dtype),
        ],
        compiler_params=pltpu.CompilerParams(
            vmem_limit_bytes=100 * 1024 * 1024,
            collective_id=0,
        ),
    )(x)
    return out


# device time: 680368 ns/iter; 1.0212x vs baseline; 1.0175x over previous
import jax
import jax.numpy as jnp
from jax import lax
from jax.experimental import pallas as pl
from jax.experimental.pallas import tpu as pltpu


M = 32768
N = 1024
Q = M // 4

ACH = (256, 256, 512, 1024, 2048, 2048, 2048)
AOFF = tuple(sum(ACH[:i]) for i in range(len(ACH)))

DX = ((0, 1536), (1536, 1536))
CY = ((3072, 1024), (4096, 1536))
CZ = ((5632, 512), (6144, 2048))
CY_AFTER = {4: CY[0], 5: CY[1]}
CZ_AFTER = {5: CZ[0], 6: CZ[1]}

N_SEMS = 2 * len(ACH) + len(ACH) + len(DX) + len(CY) + len(CZ)
MAX_ADD_ROWS = 3072


def _body(x_ref, out_ref, rem_ref, recv_sems, send_sems, lsems,
          vx_ref, vr_ref, vo_ref):
    my_x = lax.axis_index("x")
    my_y = lax.axis_index("y")
    my_z = lax.axis_index("z")
    xp = (1 - my_x, my_y, my_z)
    yp = (my_x, 1 - my_y, my_z)
    zp = (my_x, my_y, 1 - my_z)

    q = 2 * my_y + my_z
    qy = 2 * (1 - my_y) + my_z
    qz = 2 * my_y + (1 - my_z)
    d = 2 * (1 - my_y) + (1 - my_z)

    sem_counter = [0]

    def mk(src_base, row, rows, dev):
        i = sem_counter[0]
        sem_counter[0] += 1
        return pltpu.make_async_remote_copy(
            src_ref=src_base.at[pl.ds(row, rows), :],
            dst_ref=rem_ref.at[pl.ds(row, rows), :],
            send_sem=send_sems.at[i],
            recv_sem=recv_sems.at[i],
            device_id=dev,
            device_id_type=pl.DeviceIdType.MESH,
        )

    def add_chunk(row, rows):
        cx = pltpu.make_async_copy(
            x_ref.at[pl.ds(row, rows), :], vx_ref.at[pl.ds(0, rows), :],
            lsems.at[0])
        cr = pltpu.make_async_copy(
            rem_ref.at[pl.ds(row, rows), :], vr_ref.at[pl.ds(0, rows), :],
            lsems.at[1])
        cx.start()
        cr.start()
        cx.wait()
        cr.wait()
        vo_ref[:rows, :] = vx_ref[:rows, :] + vr_ref[:rows, :]
        co = pltpu.make_async_copy(
            vo_ref.at[pl.ds(0, rows), :], out_ref.at[pl.ds(row, rows), :],
            lsems.at[2])
        co.start()
        co.wait()

    A = []
    for c in range(len(ACH)):
        r = mk(x_ref, q * Q + AOFF[c], ACH[c], xp)
        r.start()
        A.append(r)
    Adx = []
    for row, rows in DX:
        r = mk(x_ref, d * Q + row, rows, xp)
        r.start()
        Adx.append(r)

    By, Bz = [], []
    for c in range(len(ACH)):
        A[c].wait_recv()
        ry = mk(rem_ref, q * Q + AOFF[c], ACH[c], yp)
        ry.start()
        By.append(ry)
        rz = mk(rem_ref, q * Q + AOFF[c], ACH[c], zp)
        rz.start()
        Bz.append(rz)
        add_chunk(q * Q + AOFF[c], ACH[c])

    Cy, Cz = [], []
    for c in range(len(ACH)):
        Bz[c].wait_recv()
        if c in CY_AFTER:
            row, rows = CY_AFTER[c]
            r = mk(rem_ref, qz * Q + row, rows, yp)
            r.start()
            Cy.append(r)
        add_chunk(qz * Q + AOFF[c], ACH[c])
        By[c].wait_recv()
        if c in CZ_AFTER:
            row, rows = CZ_AFTER[c]
            r = mk(rem_ref, qy * Q + row, rows, zp)
            r.start()
            Cz.append(r)
        add_chunk(qy * Q + AOFF[c], ACH[c])

    for i, (row, rows) in enumerate(DX):
        Adx[i].wait_recv()
        add_chunk(d * Q + row, rows)
    for i in range(2):
        Cy[i].wait_recv()
        add_chunk(d * Q + CY[i][0], CY[i][1])
        Cz[i].wait_recv()
        add_chunk(d * Q + CZ[i][0], CZ[i][1])

    for r in A + Adx + By + Bz + Cy + Cz:
        r.wait_send()


def kernel(x):
    m, n = x.shape
    assert (m, n) == (M, N)

    out, _rem = pl.pallas_call(
        _body,
        out_shape=[
            jax.ShapeDtypeStruct((m, n), x.dtype),
            jax.ShapeDtypeStruct((m, n), x.dtype),
        ],
        in_specs=[pl.BlockSpec(memory_space=pltpu.MemorySpace.HBM)],
        out_specs=[
            pl.BlockSpec(memory_space=pltpu.MemorySpace.HBM),
            pl.BlockSpec(memory_space=pltpu.MemorySpace.HBM),
        ],
        scratch_shapes=[
            pltpu.SemaphoreType.DMA((N_SEMS,)),
            pltpu.SemaphoreType.DMA((N_SEMS,)),
            pltpu.SemaphoreType.DMA((3,)),
            pltpu.VMEM((MAX_ADD_ROWS, N), x.dtype),
            pltpu.VMEM((MAX_ADD_ROWS, N), x.dtype),
            pltpu.VMEM((MAX_ADD_ROWS, N), x.dtype),
        ],
        compiler_params=pltpu.CompilerParams(
            vmem_limit_bytes=100 * 1024 * 1024,
        ),
    )(x)
    return out
